# baseline (speedup 1.0000x reference)
import functools

import jax
import jax.numpy as jnp
from jax import lax
from jax.experimental import pallas as pl
from jax.experimental.pallas import tpu as pltpu
from jax.experimental.pallas import tpu_sc as plsc

EMB = 16
B = 16384
CTX = 20

NC = 2
NS = 16
NW = NC * NS
BPW = B // NW
CB = 256
NCHUNK = BPW // CB
BLK = 128
NBLK = CB * CTX // BLK
WBLK = CB // BLK
GROUPS = CB // 16


def _sc_scores(pos_u, pos_w, neg_u, neg_w, U, W):
    mesh = plsc.VectorSubcoreMesh(core_axis_name="c", subcore_axis_name="s")

    @functools.partial(
        pl.kernel,
        out_type=(
            jax.ShapeDtypeStruct((B,), jnp.float32),
            jax.ShapeDtypeStruct((B,), jnp.float32),
        ),
        mesh=mesh,
        compiler_params=pltpu.CompilerParams(
            needs_layout_passes=False, use_tc_tiling_on_sc=False),
        scratch_types=[
            pltpu.VMEM((NBLK, BLK), jnp.int32),
            pltpu.VMEM((CB * CTX, EMB), jnp.float32),
            pltpu.VMEM((CB,), jnp.int32),
            pltpu.VMEM((CB, EMB), jnp.float32),
            pltpu.VMEM((CB * EMB,), jnp.float32),
            pltpu.VMEM((CB,), jnp.float32),
            pltpu.SemaphoreType.DMA,
        ],
    )
    def k(pos_u_h, pos_w_h, neg_u_h, neg_w_h, U_h, W_h, out_p, out_n,
          uidx, rows, widx, wrows, prod, scores, sem):
        wid = lax.axis_index("s") * NC + lax.axis_index("c")
        base = wid * BPW
        for side in range(2):
            uh = (pos_u_h, neg_u_h)[side]
            wh = (pos_w_h, neg_w_h)[side]
            oh = (out_p, out_n)[side]
            for ci in range(NCHUNK):
                gb = base + ci * CB
                urow = pl.multiple_of(gb * CTX // BLK, 8)
                pltpu.sync_copy(uh.at[pl.ds(urow, NBLK)], uidx)
                pltpu.sync_copy(wh.at[pl.ds(pl.multiple_of(gb, CB), CB)], widx)

                def fire(j, _):
                    pltpu.async_copy(
                        U_h.at[uidx.at[j]], rows.at[pl.ds(j * BLK, BLK)], sem)
                    return 0

                lax.fori_loop(0, NBLK, fire, 0)

                def firew(j, _):
                    pltpu.async_copy(
                        W_h.at[widx.at[pl.ds(j * BLK, BLK)]],
                        wrows.at[pl.ds(j * BLK, BLK)], sem)
                    return 0

                lax.fori_loop(0, WBLK, firew, 0)

                def drain(j, _):
                    pltpu.make_async_copy(
                        U_h.at[uidx.at[0]], rows.at[pl.ds(0, BLK)], sem).wait()
                    return 0

                lax.fori_loop(0, NBLK, drain, 0)

                def drainw(j, _):
                    pltpu.make_async_copy(
                        W_h.at[widx.at[pl.ds(0, BLK)]],
                        wrows.at[pl.ds(0, BLK)], sem).wait()
                    return 0

                lax.fori_loop(0, WBLK, drainw, 0)

                def bag(b, _):
                    r0 = b * CTX
                    acc = rows[r0, :]
                    for e in range(1, CTX):
                        acc = acc + rows[r0 + e, :]
                    prod[pl.ds(pl.multiple_of(b * EMB, 8), EMB)] = acc * wrows[b, :]
                    return 0

                lax.fori_loop(0, CB, bag, 0)

                def dot(g, _):
                    bvec = jnp.int32(16) * g + lax.iota(jnp.int32, 16)
                    fvec = bvec * EMB
                    sacc = jnp.zeros((16,), jnp.float32)
                    for e in range(EMB):
                        sacc = sacc + plsc.load_gather(prod, [fvec + e])
                    scores[pl.ds(g * 16, 16)] = sacc
                    return 0

                lax.fori_loop(0, GROUPS, dot, 0)
                pltpu.sync_copy(scores, oh.at[pl.ds(pl.multiple_of(gb, CB), CB)])

    return k(pos_u, pos_w, neg_u, neg_w, U, W)


def _loss_tc(sp, sn):
    def body(sp_ref, sn_ref, out_ref):
        pos = sp_ref[...]
        neg = sn_ref[...]
        lp = jnp.minimum(pos, 0.0) - jnp.log1p(jnp.exp(-jnp.abs(pos)))
        ln = jnp.minimum(-neg, 0.0) - jnp.log1p(jnp.exp(-jnp.abs(neg)))
        out_ref[0, 0] = -(jnp.sum(lp) + jnp.sum(ln))

    out = pl.pallas_call(
        body,
        out_shape=jax.ShapeDtypeStruct((1, 1), jnp.float32),
        out_specs=pl.BlockSpec(memory_space=pltpu.SMEM),
    )(sp.reshape(128, 128), sn.reshape(128, 128))
    return out[0, 0]


def kernel(pos_u, pos_w, neg_u, neg_w, U, W):
    pu = jnp.asarray(pos_u, jnp.int32).reshape(B * CTX // BLK, BLK)
    nu = jnp.asarray(neg_u, jnp.int32).reshape(B * CTX // BLK, BLK)
    pw = jnp.asarray(pos_w, jnp.int32)
    nw = jnp.asarray(neg_w, jnp.int32)
    sp, sn = _sc_scores(pu, pw, nu, nw, U, W)
    return _loss_tc(sp, sn)


# speedup vs baseline: 1.0002x; 1.0002x over previous
import functools

import jax
import jax.numpy as jnp
from jax import lax
from jax.experimental import pallas as pl
from jax.experimental.pallas import tpu as pltpu
from jax.experimental.pallas import tpu_sc as plsc

EMB = 16
B = 16384
CTX = 20

NC = 2
NS = 16
NW = NC * NS
BPW = B // NW
CB = 256
NCHUNK = BPW // CB
BLK = 128
NBLK = CB * CTX // BLK
WBLK = CB // BLK
GROUPS = CB // 16


def _sc_scores(pos_u, pos_w, neg_u, neg_w, U, W):
    mesh = plsc.VectorSubcoreMesh(core_axis_name="c", subcore_axis_name="s")

    @functools.partial(
        pl.kernel,
        out_type=(
            jax.ShapeDtypeStruct((B,), jnp.float32),
            jax.ShapeDtypeStruct((B,), jnp.float32),
        ),
        mesh=mesh,
        compiler_params=pltpu.CompilerParams(
            needs_layout_passes=False, use_tc_tiling_on_sc=False),
        scratch_types=[
            pltpu.VMEM((CB * CTX,), jnp.int32),
            pltpu.VMEM((CB * CTX, EMB), jnp.float32),
            pltpu.VMEM((CB,), jnp.int32),
            pltpu.VMEM((CB, EMB), jnp.float32),
            pltpu.VMEM((CB * EMB,), jnp.float32),
            pltpu.VMEM((CB,), jnp.float32),
            pltpu.SemaphoreType.DMA,
        ],
    )
    def k(pos_u_h, pos_w_h, neg_u_h, neg_w_h, U_h, W_h, out_p, out_n,
          uidx, rows, widx, wrows, prod, scores, sem):
        wid = lax.axis_index("s") * NC + lax.axis_index("c")
        base = wid * BPW
        for side in range(2):
            uh = (pos_u_h, neg_u_h)[side]
            wh = (pos_w_h, neg_w_h)[side]
            oh = (out_p, out_n)[side]
            for ci in range(NCHUNK):
                gb = base + ci * CB
                pltpu.sync_copy(
                    uh.at[pl.ds(pl.multiple_of(gb * CTX, 8), CB * CTX)], uidx)
                pltpu.sync_copy(wh.at[pl.ds(pl.multiple_of(gb, CB), CB)], widx)
                cu = pltpu.async_copy(U_h.at[uidx], rows, sem)
                cw = pltpu.async_copy(W_h.at[widx], wrows, sem)
                cu.wait()
                cw.wait()

                def bag(b, _):
                    r0 = b * CTX
                    acc = rows[r0, :]
                    for e in range(1, CTX):
                        acc = acc + rows[r0 + e, :]
                    prod[pl.ds(pl.multiple_of(b * EMB, 8), EMB)] = acc * wrows[b, :]
                    return 0

                lax.fori_loop(0, CB, bag, 0)

                def dot(g, _):
                    bvec = jnp.int32(16) * g + lax.iota(jnp.int32, 16)
                    fvec = bvec * EMB
                    sacc = jnp.zeros((16,), jnp.float32)
                    for e in range(EMB):
                        sacc = sacc + plsc.load_gather(prod, [fvec + e])
                    scores[pl.ds(g * 16, 16)] = sacc
                    return 0

                lax.fori_loop(0, GROUPS, dot, 0)
                pltpu.sync_copy(scores, oh.at[pl.ds(pl.multiple_of(gb, CB), CB)])

    return k(pos_u, pos_w, neg_u, neg_w, U, W)


def _loss_tc(sp, sn):
    def body(sp_ref, sn_ref, out_ref):
        pos = sp_ref[...]
        neg = sn_ref[...]
        lp = jnp.minimum(pos, 0.0) - jnp.log1p(jnp.exp(-jnp.abs(pos)))
        ln = jnp.minimum(-neg, 0.0) - jnp.log1p(jnp.exp(-jnp.abs(neg)))
        out_ref[0, 0] = -(jnp.sum(lp) + jnp.sum(ln))

    out = pl.pallas_call(
        body,
        out_shape=jax.ShapeDtypeStruct((1, 1), jnp.float32),
        out_specs=pl.BlockSpec(memory_space=pltpu.SMEM),
    )(sp.reshape(128, 128), sn.reshape(128, 128))
    return out[0, 0]


def kernel(pos_u, pos_w, neg_u, neg_w, U, W):
    pu = jnp.asarray(pos_u, jnp.int32).reshape(B * CTX)
    nu = jnp.asarray(neg_u, jnp.int32).reshape(B * CTX)
    pw = jnp.asarray(pos_w, jnp.int32)
    nw = jnp.asarray(neg_w, jnp.int32)
    sp, sn = _sc_scores(pu, pw, nu, nw, U, W)
    return _loss_tc(sp, sn)
